# baseline (speedup 1.0000x reference)
import functools

import jax
import jax.numpy as jnp
from jax import lax
from jax.experimental import pallas as pl
from jax.experimental.pallas import tpu as pltpu

B = 1024
N = 100000
VB = 6144
NB = (N + VB - 1) // VB
GR = 64
CH = 16
UNROLL = 4


def _gather_body(t_ref, *refs):
    out_ref = refs[-1]
    j = pl.program_id(0)
    sub = lax.broadcasted_iota(jnp.int32, (8, 128), 0)
    lane = lax.broadcasted_iota(jnp.int32, (8, 128), 1)
    for m in range(GR):
        r = j * GR + m
        t = t_ref[r]
        hit = (sub == lax.rem(t, 8)) & (lane == lax.rem(r, 128))
        x = refs[m][...]
        vm = jnp.sum(jnp.where(hit, x, 0.0), axis=(0, 1), keepdims=True)
        out_ref[pl.ds(m, 1), :] = vm


def _tile_imap(m, j, t_ref):
    r = j * GR + m
    return (t_ref[r] // 8, r // 128)


def _gather(outputs_t, t32):
    grid_spec = pltpu.PrefetchScalarGridSpec(
        num_scalar_prefetch=1,
        grid=(B // GR,),
        in_specs=[
            pl.BlockSpec((8, 128), functools.partial(_tile_imap, m))
            for m in range(GR)
        ],
        out_specs=pl.BlockSpec((GR, 1), lambda j, t_ref: (j, 0)),
    )
    return pl.pallas_call(
        _gather_body,
        grid_spec=grid_spec,
        out_shape=jax.ShapeDtypeStruct((B, 1), jnp.float32),
    )(t32, *([outputs_t] * GR))


def _count_body(x_ref, v_ref, t_ref, o1_ref, o5_ref, acc_ref):
    j = pl.program_id(0)

    pad = NB * VB - N
    if pad:
        @pl.when(j == NB - 1)
        def _():
            x_ref[pl.ds(VB - pad, pad), :] = jnp.full(
                (pad, B), -jnp.inf, jnp.float32)

    v = v_ref[...]
    t = t_ref[...]
    tl = t - j * VB
    iota = lax.broadcasted_iota(jnp.int32, (CH, B), 0)

    def chunk(c, acc):
        x = x_ref[pl.ds(c * CH, CH), :]
        sub = iota + c * CH
        beats = (x > v) | ((x == v) & (sub < tl))
        return acc + beats.astype(jnp.int32)

    psum = lax.fori_loop(0, VB // CH, chunk, jnp.zeros((CH, B), jnp.int32),
                         unroll=UNROLL)
    cnt = jnp.sum(psum, axis=0, keepdims=True)

    @pl.when(j == 0)
    def _():
        acc_ref[...] = cnt

    @pl.when(j > 0)
    def _():
        acc_ref[...] = acc_ref[...] + cnt

    @pl.when(j == NB - 1)
    def _():
        rank = acc_ref[...]
        scale = jnp.float32(100.0 / B)
        o1_ref[...] = jnp.sum((rank < 1).astype(jnp.float32), axis=1,
                              keepdims=True) * scale
        o5_ref[...] = jnp.sum((rank < 5).astype(jnp.float32), axis=1,
                              keepdims=True) * scale


def _tc_count(outputs_t, v_row, t_row, interpret=False):
    out1, out5 = pl.pallas_call(
        _count_body,
        grid=(NB,),
        in_specs=[
            pl.BlockSpec((VB, B), lambda j: (j, 0)),
            pl.BlockSpec((1, B), lambda j: (0, 0)),
            pl.BlockSpec((1, B), lambda j: (0, 0)),
        ],
        out_specs=[
            pl.BlockSpec((1, 1), lambda j: (0, 0)),
            pl.BlockSpec((1, 1), lambda j: (0, 0)),
        ],
        out_shape=[
            jax.ShapeDtypeStruct((1, 1), jnp.float32),
            jax.ShapeDtypeStruct((1, 1), jnp.float32),
        ],
        scratch_shapes=[pltpu.VMEM((1, B), jnp.int32)],
        interpret=interpret,
    )(outputs_t, v_row, t_row)
    return out1, out5


def kernel(outputs, targets):
    t32 = targets.astype(jnp.int32)
    xt = outputs.T
    v = _gather(xt, t32)
    out1, out5 = _tc_count(xt, v.reshape(1, B), t32.reshape(1, B))
    return (out1.reshape(1), out5.reshape(1))


# speedup vs baseline: 1.1463x; 1.1463x over previous
import functools

import jax
import jax.numpy as jnp
from jax import lax
from jax.experimental import pallas as pl
from jax.experimental.pallas import tpu as pltpu

B = 1024
N = 100000
VB = 4096
NB = (N + VB - 1) // VB
GR = 64
CH = 16
UNROLL = 4


def _gather_body(x_any, t_smem, out_ref, win, sem):
    def issue(i, c):
        t = t_smem[0, i]
        r0 = pl.multiple_of((t // 8) * 8, 8)
        l0 = pl.multiple_of((i // 128) * 128, 128)
        pltpu.make_async_copy(
            x_any.at[pl.ds(r0, 8), pl.ds(l0, 128)], win.at[i], sem).start()
        return c

    lax.fori_loop(0, B, issue, 0)

    def drain(i, c):
        pltpu.make_async_copy(
            x_any.at[pl.ds(0, 8), pl.ds(0, 128)], win.at[i], sem).wait()
        return c

    lax.fori_loop(0, B, drain, 0)

    sub = lax.broadcasted_iota(jnp.int32, (8, 128), 0)
    lane = lax.broadcasted_iota(jnp.int32, (8, 128), 1)
    for g in range(B // 128):
        def pick(i, accv):
            idx = g * 128 + i
            t = t_smem[0, idx]
            hit = (sub == lax.rem(t, 8)) & (lane == i)
            return accv + jnp.where(hit, win[idx], 0.0)

        accv = lax.fori_loop(0, 128, pick, jnp.zeros((8, 128), jnp.float32))
        out_ref[pl.ds(0, 1), pl.ds(g * 128, 128)] = jnp.sum(
            accv, axis=0, keepdims=True)


def _gather(outputs_t, t_row):
    return pl.pallas_call(
        _gather_body,
        in_specs=[
            pl.BlockSpec(memory_space=pl.ANY),
            pl.BlockSpec(memory_space=pltpu.SMEM),
        ],
        out_specs=pl.BlockSpec((1, B), lambda: (0, 0)),
        out_shape=jax.ShapeDtypeStruct((1, B), jnp.float32),
        scratch_shapes=[
            pltpu.VMEM((B, 8, 128), jnp.float32),
            pltpu.SemaphoreType.DMA,
        ],
    )(outputs_t, t_row)


def _count_body(x_ref, v_ref, t_ref, o1_ref, o5_ref, acc_ref):
    j = pl.program_id(0)

    pad = NB * VB - N
    if pad:
        @pl.when(j == NB - 1)
        def _():
            x_ref[pl.ds(VB - pad, pad), :] = jnp.full(
                (pad, B), -jnp.inf, jnp.float32)

    v = v_ref[...]
    t = t_ref[...]
    tl = t - j * VB
    iota = lax.broadcasted_iota(jnp.int32, (CH, B), 0)

    def chunk(c, acc):
        x = x_ref[pl.ds(c * CH, CH), :]
        sub = iota + c * CH
        beats = (x > v) | ((x == v) & (sub < tl))
        return acc + beats.astype(jnp.int32)

    psum = lax.fori_loop(0, VB // CH, chunk, jnp.zeros((CH, B), jnp.int32),
                         unroll=UNROLL)
    cnt = jnp.sum(psum, axis=0, keepdims=True)

    @pl.when(j == 0)
    def _():
        acc_ref[...] = cnt

    @pl.when(j > 0)
    def _():
        acc_ref[...] = acc_ref[...] + cnt

    @pl.when(j == NB - 1)
    def _():
        rank = acc_ref[...]
        scale = jnp.float32(100.0 / B)
        o1_ref[...] = jnp.sum((rank < 1).astype(jnp.float32), axis=1,
                              keepdims=True) * scale
        o5_ref[...] = jnp.sum((rank < 5).astype(jnp.float32), axis=1,
                              keepdims=True) * scale


def _tc_count(outputs_t, v_row, t_row, interpret=False):
    out1, out5 = pl.pallas_call(
        _count_body,
        grid=(NB,),
        in_specs=[
            pl.BlockSpec((VB, B), lambda j: (j, 0)),
            pl.BlockSpec((1, B), lambda j: (0, 0)),
            pl.BlockSpec((1, B), lambda j: (0, 0)),
        ],
        out_specs=[
            pl.BlockSpec((1, 1), lambda j: (0, 0)),
            pl.BlockSpec((1, 1), lambda j: (0, 0)),
        ],
        out_shape=[
            jax.ShapeDtypeStruct((1, 1), jnp.float32),
            jax.ShapeDtypeStruct((1, 1), jnp.float32),
        ],
        scratch_shapes=[pltpu.VMEM((1, B), jnp.int32)],
        interpret=interpret,
    )(outputs_t, v_row, t_row)
    return out1, out5


def kernel(outputs, targets):
    t_row = targets.astype(jnp.int32).reshape(1, B)
    xt = outputs.T
    v = _gather(xt, t_row)
    out1, out5 = _tc_count(xt, v, t_row)
    return (out1.reshape(1), out5.reshape(1))
